# baseline (speedup 1.0000x reference)
import functools

import jax
import jax.numpy as jnp
from jax import lax
from jax.experimental import pallas as pl
from jax.experimental.pallas import tpu as pltpu
from jax.experimental.pallas import tpu_sc as plsc

_B = 4096
_N = 20
_D = 64
_NC = 2
_NS = 16
_NW = _NC * _NS
_BPW = _B // _NW
_CHUNK = 32
_NCHUNK = _BPW // _CHUNK


def _make_sc_kernel():
    mesh = plsc.VectorSubcoreMesh(core_axis_name="c", subcore_axis_name="s")

    @functools.partial(
        pl.kernel,
        out_type=jax.ShapeDtypeStruct((_N, _B), jnp.float32),
        mesh=mesh,
        scratch_types=[
            pltpu.VMEM((_N, _CHUNK), jnp.int32),
            pltpu.VMEM((_N * _CHUNK, _D), jnp.float32),
            pltpu.VMEM((_D, _CHUNK), jnp.float32),
            pltpu.VMEM((_N, _CHUNK), jnp.float32),
            pltpu.SemaphoreType.DMA,
        ],
        compiler_params=pltpu.CompilerParams(
            use_tc_tiling_on_sc=False,
            needs_layout_passes=False,
        ),
    )
    def sc_kernel(ctx_hbm, ids_hbm, table_hbm, out_hbm,
                  idx_v, rows_v, ctx_v, out_v, sem):
        wid = lax.axis_index("s") * _NC + lax.axis_index("c")
        lanes = lax.iota(jnp.int32, 16)

        for g in range(_NCHUNK):
            b0 = wid * _BPW + g * _CHUNK
            pltpu.sync_copy(ids_hbm.at[:, pl.ds(b0, _CHUNK)], idx_v)
            copies = [
                pltpu.async_copy(
                    table_hbm.at[idx_v.at[n]],
                    rows_v.at[pl.ds(n * _CHUNK, _CHUNK)],
                    sem,
                )
                for n in range(_N)
            ]
            pltpu.sync_copy(ctx_hbm.at[:, pl.ds(b0, _CHUNK)], ctx_v)
            for cpy in copies:
                cpy.wait()

            for h in range(_CHUNK // 16):
                rowvecs = [
                    lanes + (n * _CHUNK + h * 16) for n in range(_N)
                ]

                def dbody(d, accs, h=h, rowvecs=rowvecs):
                    col = lax.broadcast_in_dim(d, (16,), ())
                    cvec = ctx_v[d, pl.ds(h * 16, 16)]
                    return tuple(
                        accs[n]
                        + plsc.load_gather(rows_v, [rowvecs[n], col]) * cvec
                        for n in range(_N)
                    )

                accs = lax.fori_loop(
                    0, _D, dbody,
                    tuple(jnp.zeros((16,), jnp.float32) for _ in range(_N)),
                )

                m = accs[0]
                for n in range(1, _N):
                    m = jnp.maximum(m, accs[n])
                es = [jnp.exp(a - m) for a in accs]
                tot = es[0]
                for n in range(1, _N):
                    tot = tot + es[n]
                for n in range(_N):
                    out_v[n, pl.ds(h * 16, 16)] = es[n] / tot

            pltpu.sync_copy(out_v, out_hbm.at[:, pl.ds(b0, _CHUNK)])

    return sc_kernel


_SC_KERNEL = _make_sc_kernel()


def kernel(context_encoded, entity_ids, entity_embeddings):
    out_t = _SC_KERNEL(context_encoded.T, entity_ids.T, entity_embeddings)
    return out_t.T


# speedup vs baseline: 1.0033x; 1.0033x over previous
import functools

import jax
import jax.numpy as jnp
from jax import lax
from jax.experimental import pallas as pl
from jax.experimental.pallas import tpu as pltpu
from jax.experimental.pallas import tpu_sc as plsc

_B = 4096
_N = 20
_D = 64
_NC = 2
_NS = 16
_NW = _NC * _NS
_BPW = _B // _NW
_CHUNK = 32
_NCHUNK = _BPW // _CHUNK
_NGRP = 10


def _make_sc_kernel():
    mesh = plsc.VectorSubcoreMesh(core_axis_name="c", subcore_axis_name="s")

    @functools.partial(
        pl.kernel,
        out_type=jax.ShapeDtypeStruct((_N, _B), jnp.float32),
        mesh=mesh,
        scratch_types=[
            pltpu.VMEM((_N, _BPW), jnp.int32),
            pltpu.VMEM((_N * _CHUNK,), jnp.int32),
            pltpu.VMEM((_N * _CHUNK, 128), jnp.float32),
            pltpu.VMEM((_D, _BPW), jnp.float32),
            pltpu.VMEM((_N, _BPW), jnp.float32),
            pltpu.SemaphoreType.DMA,
        ],
        compiler_params=pltpu.CompilerParams(
            use_tc_tiling_on_sc=True,
            needs_layout_passes=False,
        ),
    )
    def sc_kernel(ctx_hbm, ids_hbm, table_hbm, out_hbm,
                  idx_v, idx2_v, rows_v, ctx_v, out_v, sem):
        wid = lax.axis_index("s") * _NC + lax.axis_index("c")
        w0 = wid * _BPW
        lanes = lax.iota(jnp.int32, 16)

        pltpu.sync_copy(ids_hbm.at[:, pl.ds(w0, _BPW)], idx_v)
        pltpu.sync_copy(ctx_hbm.at[:, pl.ds(w0, _BPW)], ctx_v)

        for c in range(_NCHUNK):
            for n in range(_N):
                for k in range(2):
                    v = idx_v[n, pl.ds(c * _CHUNK + k * 16, 16)]
                    idx2_v[pl.ds(n * _CHUNK + k * 16, 16)] = (
                        lax.shift_right_logical(v, 1)
                    )
            copies = [
                pltpu.async_copy(
                    table_hbm.at[idx2_v.at[pl.ds(n * _CHUNK, _CHUNK)]],
                    rows_v.at[pl.ds(n * _CHUNK, _CHUNK)],
                    sem,
                )
                for n in range(_N)
            ]
            for cpy in copies:
                cpy.wait()

            for h in range(_CHUNK // 16):
                bcol = c * _CHUNK + h * 16
                for g0 in range(0, _N, _NGRP):
                    grp = range(g0, g0 + _NGRP)
                    rowv = [lanes + (n * _CHUNK + h * 16) for n in grp]
                    colb = [
                        lax.shift_left(idx_v[n, pl.ds(bcol, 16)] & 1, 6)
                        for n in grp
                    ]

                    def dbody(d, accs, rowv=rowv, colb=colb):
                        cvec = ctx_v[d, pl.ds(bcol, 16)]
                        dcast = lax.broadcast_in_dim(d, (16,), ())
                        return tuple(
                            accs[i]
                            + plsc.load_gather(
                                rows_v, [rowv[i], colb[i] + dcast]
                            ) * cvec
                            for i in range(_NGRP)
                        )

                    accs = lax.fori_loop(
                        0, _D, dbody,
                        tuple(jnp.zeros((16,), jnp.float32)
                              for _ in range(_NGRP)),
                    )
                    for i, n in enumerate(grp):
                        out_v[n, pl.ds(bcol, 16)] = accs[i]

                m = out_v[0, pl.ds(bcol, 16)]
                for n in range(1, _N):
                    m = jnp.maximum(m, out_v[n, pl.ds(bcol, 16)])
                tot = jnp.zeros((16,), jnp.float32)
                for n in range(_N):
                    e = jnp.exp(out_v[n, pl.ds(bcol, 16)] - m)
                    tot = tot + e
                    out_v[n, pl.ds(bcol, 16)] = e
                for n in range(_N):
                    out_v[n, pl.ds(bcol, 16)] = (
                        out_v[n, pl.ds(bcol, 16)] / tot
                    )

        pltpu.sync_copy(out_v, out_hbm.at[:, pl.ds(w0, _BPW)])

    return sc_kernel


_SC_KERNEL = _make_sc_kernel()


def kernel(context_encoded, entity_ids, entity_embeddings):
    table2 = entity_embeddings.reshape(entity_embeddings.shape[0] // 2, 128)
    out_t = _SC_KERNEL(context_encoded.T, entity_ids.T, table2)
    return out_t.T
